# Initial kernel scaffold; baseline (speedup 1.0000x reference)
import jax
import jax.numpy as jnp
from jax.experimental import pallas as pl


def kernel(x, table):
    raise NotImplementedError("write your pallas kernel here")


# speedup vs baseline: 1.1021x; 1.1021x over previous
import functools

import jax
import jax.numpy as jnp
from jax import lax
from jax.experimental import pallas as pl
from jax.experimental.pallas import tpu as pltpu
from jax.experimental.pallas import tpu_sc as plsc

D = 32
NC, NS = 2, 16
NW = NC * NS
ROWS_PER_GATHER = 128
GATHERS_PER_CHUNK = 8
CHUNK_ROWS = ROWS_PER_GATHER * GATHERS_PER_CHUNK


@functools.partial(jax.jit, static_argnames=())
def _embed(table, idx3):
    n_idx = idx3.shape[1]
    n_chunks = n_idx // GATHERS_PER_CHUNK
    rows_per_w = n_idx * ROWS_PER_GATHER
    B = NW * rows_per_w
    mesh = plsc.VectorSubcoreMesh(core_axis_name="c", subcore_axis_name="s")

    @functools.partial(
        pl.kernel,
        mesh=mesh,
        compiler_params=pltpu.CompilerParams(use_tc_tiling_on_sc=False),
        out_type=jax.ShapeDtypeStruct((B, D), jnp.float32),
        scratch_types=[
            pltpu.VMEM((n_idx, ROWS_PER_GATHER), jnp.int32),
            pltpu.VMEM((CHUNK_ROWS, D), jnp.float32),
            pltpu.SemaphoreType.DMA,
        ],
    )
    def k(table_hbm, idx_hbm, out_hbm, idx_v, rows_v, sem):
        wid = lax.axis_index("s") * NC + lax.axis_index("c")
        pltpu.sync_copy(idx_hbm.at[wid], idx_v)
        base = wid * rows_per_w

        def chunk_body(c, carry):
            copies = []
            for g in range(GATHERS_PER_CHUNK):
                cp = pltpu.async_copy(
                    table_hbm.at[idx_v.at[c * GATHERS_PER_CHUNK + g]],
                    rows_v.at[pl.ds(g * ROWS_PER_GATHER, ROWS_PER_GATHER)],
                    sem,
                )
                copies.append(cp)
            for cp in copies:
                cp.wait()
            pltpu.sync_copy(
                rows_v, out_hbm.at[pl.ds(base + c * CHUNK_ROWS, CHUNK_ROWS)]
            )
            return carry

        lax.fori_loop(0, n_chunks, chunk_body, None)

    return k(table, idx3)


def kernel(x, table):
    bsz, hist = x.shape
    flat = bsz * hist
    assert flat % (NW * ROWS_PER_GATHER) == 0
    idx3 = x.astype(jnp.int32).reshape(NW, flat // (NW * ROWS_PER_GATHER),
                                       ROWS_PER_GATHER)
    out = _embed(table, idx3)
    return out.reshape(bsz, hist, D)


# speedup vs baseline: 1.1092x; 1.0064x over previous
import functools

import jax
import jax.numpy as jnp
from jax import lax
from jax.experimental import pallas as pl
from jax.experimental.pallas import tpu as pltpu
from jax.experimental.pallas import tpu_sc as plsc

D = 32
NC, NS = 2, 16
NW = NC * NS
ROWS_PER_GATHER = 128
GATHERS_PER_CHUNK = 10
CHUNK_ROWS = ROWS_PER_GATHER * GATHERS_PER_CHUNK


@functools.partial(jax.jit, static_argnames=())
def _embed(table, idx3):
    n_idx = idx3.shape[1]
    n_pairs = n_idx // (2 * GATHERS_PER_CHUNK)
    rows_per_w = n_idx * ROWS_PER_GATHER
    B = NW * rows_per_w
    mesh = plsc.VectorSubcoreMesh(core_axis_name="c", subcore_axis_name="s")

    @functools.partial(
        pl.kernel,
        mesh=mesh,
        compiler_params=pltpu.CompilerParams(use_tc_tiling_on_sc=False),
        out_type=jax.ShapeDtypeStruct((B, D), jnp.float32),
        scratch_types=[
            pltpu.VMEM((n_idx, ROWS_PER_GATHER), jnp.int32),
            pltpu.VMEM((CHUNK_ROWS, D), jnp.float32),
            pltpu.VMEM((CHUNK_ROWS, D), jnp.float32),
            pltpu.SemaphoreType.DMA,
            pltpu.SemaphoreType.DMA,
        ],
    )
    def k(table_hbm, idx_hbm, out_hbm, idx_v, rows_a, rows_b, sem_a, sem_b):
        wid = lax.axis_index("s") * NC + lax.axis_index("c")
        pltpu.sync_copy(idx_hbm.at[wid], idx_v)
        base = wid * rows_per_w

        def fire(c, buf, sem):
            copies = []
            for g in range(GATHERS_PER_CHUNK):
                cp = pltpu.async_copy(
                    table_hbm.at[idx_v.at[c * GATHERS_PER_CHUNK + g]],
                    buf.at[pl.ds(g * ROWS_PER_GATHER, ROWS_PER_GATHER)],
                    sem,
                )
                copies.append(cp)
            return copies

        def drain_and_store(copies, buf, c):
            for cp in copies:
                cp.wait()
            pltpu.sync_copy(
                buf, out_hbm.at[pl.ds(base + c * CHUNK_ROWS, CHUNK_ROWS)]
            )

        def pair_body(t, carry):
            ca = fire(2 * t, rows_a, sem_a)
            cb = fire(2 * t + 1, rows_b, sem_b)
            drain_and_store(ca, rows_a, 2 * t)
            drain_and_store(cb, rows_b, 2 * t + 1)
            return carry

        lax.fori_loop(0, n_pairs, pair_body, None)

    return k(table, idx3)


def kernel(x, table):
    bsz, hist = x.shape
    flat = bsz * hist
    assert flat % (NW * ROWS_PER_GATHER) == 0
    idx3 = x.astype(jnp.int32).reshape(NW, flat // (NW * ROWS_PER_GATHER),
                                       ROWS_PER_GATHER)
    out = _embed(table, idx3)
    return out.reshape(bsz, hist, D)
